# baseline (speedup 1.0000x reference)
import jax
import jax.numpy as jnp
from jax import lax
from jax.experimental import pallas as pl
from jax.experimental.pallas import tpu as pltpu
from jax.experimental.pallas import tpu_sc as plsc

B, T, D = 4, 8192, 768
NC, NS = 2, 16
NW = NC * NS
PW = T // NW
CPOS = 16
NCHUNK = PW // CPOS
NPAIR = NCHUNK // 2
NSL = D // 16
NSLOT = 8


def _body(x_hbm, w_hbm, out_hbm, refs, sems):
    xb = refs[:NSLOT]
    wb = refs[NSLOT:]
    lsem = sems[:NSLOT]
    ssem = sems[NSLOT:2 * NSLOT]
    wsem = sems[2 * NSLOT:]
    wid = lax.axis_index("s") * NC + lax.axis_index("c")
    base = wid * PW

    def xsl(b, c):
        return x_hbm.at[b, pl.ds(base + c * CPOS, CPOS), :]

    def osl(b, c):
        return out_hbm.at[b, pl.ds(base + c * CPOS, CPOS), :]

    def wslc(c):
        return w_hbm.at[pl.ds(base + c * CPOS, CPOS), :]

    def emit_pair(pair, first, last):
        c0 = 2 * pair
        for u in range(NSLOT):
            b = u % B
            half = u // B
            c = c0 + half
            wbuf = wb[half]
            if u == 0:
                if not first:
                    pltpu.async_copy(wslc(c0 + 1), wb[1], wsem[1])
                pltpu.make_async_copy(wslc(c0), wb[0], wsem[0]).wait()
            if u == B:
                pltpu.make_async_copy(wslc(c0 + 1), wb[1], wsem[1]).wait()
                if not last:
                    pltpu.async_copy(wslc(c0 + 2), wb[0], wsem[0])

            pltpu.make_async_copy(xsl(b, c), xb[u], lsem[u]).wait()

            u4 = (u + 4) % NSLOT
            c4 = c + 1
            if not (last and u >= B):
                if not (first and u < B):
                    cp = c - 1
                    pltpu.make_async_copy(xb[u4], osl(b, cp), ssem[u4]).wait()
                pltpu.async_copy(xsl(b, c4), xb[u4], lsem[u4])

            @plsc.parallel_loop(0, CPOS, 1)
            def rowadd(r):
                @plsc.parallel_loop(0, D, 16, unroll=3)
                def coladd(col):
                    plsc.addupdate(xb[u].at[r, pl.ds(col, 16)],
                                   wb[half][r, pl.ds(col, 16)])

            pltpu.async_copy(xb[u], osl(b, c), ssem[u])

    pltpu.async_copy(wslc(0), wb[0], wsem[0])
    pltpu.async_copy(wslc(1), wb[1], wsem[1])
    for u in range(B):
        pltpu.async_copy(xsl(u, 0), xb[u], lsem[u])

    emit_pair(0, True, NPAIR == 1)
    if NPAIR > 2:
        def loop_body(pair, carry):
            emit_pair(pair, False, False)
            return carry
        lax.fori_loop(1, NPAIR - 1, loop_body, 0)
    if NPAIR > 1:
        emit_pair(NPAIR - 1, False, True)

    for u in range(NSLOT):
        b = u % B
        c = NCHUNK - 2 + u // B
        pltpu.make_async_copy(xb[u], osl(b, c), ssem[u]).wait()


def _kernel_body(x_hbm, w_hbm, out_hbm, *scratch):
    _body(x_hbm, w_hbm, out_hbm, scratch[:NSLOT + 2], scratch[NSLOT + 2:])


@jax.jit
def _run(x, w):
    mesh = plsc.VectorSubcoreMesh(
        core_axis_name="c", subcore_axis_name="s", num_cores=NC, num_subcores=NS
    )
    return pl.kernel(
        _kernel_body,
        out_type=jax.ShapeDtypeStruct((B, T, D), jnp.float32),
        mesh=mesh,
        scratch_types=(
            [pltpu.VMEM((CPOS, D), jnp.float32)] * (NSLOT + 2)
            + [pltpu.SemaphoreType.DMA] * (2 * NSLOT + 2)
        ),
    )(x, w)


def kernel(x, lookup_weight):
    return _run(x, lookup_weight)


# speedup vs baseline: 1.1124x; 1.1124x over previous
import jax
import jax.numpy as jnp
from jax import lax
from jax.experimental import pallas as pl
from jax.experimental.pallas import tpu as pltpu
from jax.experimental.pallas import tpu_sc as plsc

B, T, D = 4, 8192, 768
NC, NS = 2, 16
NW = NC * NS
PW = T // NW
CPOS = 16
NCHUNK = PW // CPOS
NPAIR = NCHUNK // 2
NSL = D // 16
NSLOT = 8


def _body(x_hbm, w_hbm, out_hbm, refs, sems):
    xb = refs[:NSLOT]
    wb = refs[NSLOT:]
    lsem = sems[:NSLOT]
    ssem = sems[NSLOT:2 * NSLOT]
    wsem = sems[2 * NSLOT:]
    wid = lax.axis_index("s") * NC + lax.axis_index("c")
    base = wid * PW

    def xsl(b, c):
        return x_hbm.at[b, pl.ds(base + c * CPOS, CPOS), :]

    def osl(b, c):
        return out_hbm.at[b, pl.ds(base + c * CPOS, CPOS), :]

    def wslc(c):
        return w_hbm.at[pl.ds(base + c * CPOS, CPOS), :]

    def emit_pair(pair, first, last):
        c0 = 2 * pair
        for u in range(NSLOT):
            b = u % B
            half = u // B
            c = c0 + half
            wbuf = wb[half]
            if u == 0:
                if not first:
                    pltpu.async_copy(wslc(c0 + 1), wb[1], wsem[1])
                pltpu.make_async_copy(wslc(c0), wb[0], wsem[0]).wait()
            if u == B:
                pltpu.make_async_copy(wslc(c0 + 1), wb[1], wsem[1]).wait()
                if not last:
                    pltpu.async_copy(wslc(c0 + 2), wb[0], wsem[0])

            pltpu.make_async_copy(xsl(b, c), xb[u], lsem[u]).wait()

            u4 = (u + 4) % NSLOT
            c4 = c + 1
            if not (last and u >= B):
                if not (first and u < B):
                    cp = c - 1
                    pltpu.make_async_copy(xb[u4], osl(b, cp), ssem[u4]).wait()
                pltpu.async_copy(xsl(b, c4), xb[u4], lsem[u4])

            @plsc.parallel_loop(0, CPOS, 1, unroll=2)
            def rowadd(r):
                @plsc.parallel_loop(0, D, 16, unroll=6)
                def coladd(col):
                    plsc.addupdate(xb[u].at[r, pl.ds(col, 16)],
                                   wb[half][r, pl.ds(col, 16)])

            pltpu.async_copy(xb[u], osl(b, c), ssem[u])

    pltpu.async_copy(wslc(0), wb[0], wsem[0])
    pltpu.async_copy(wslc(1), wb[1], wsem[1])
    for u in range(B):
        pltpu.async_copy(xsl(u, 0), xb[u], lsem[u])

    emit_pair(0, True, NPAIR == 1)
    if NPAIR > 2:
        def loop_body(pair, carry):
            emit_pair(pair, False, False)
            return carry
        lax.fori_loop(1, NPAIR - 1, loop_body, 0)
    if NPAIR > 1:
        emit_pair(NPAIR - 1, False, True)

    for u in range(NSLOT):
        b = u % B
        c = NCHUNK - 2 + u // B
        pltpu.make_async_copy(xb[u], osl(b, c), ssem[u]).wait()


def _kernel_body(x_hbm, w_hbm, out_hbm, *scratch):
    _body(x_hbm, w_hbm, out_hbm, scratch[:NSLOT + 2], scratch[NSLOT + 2:])


@jax.jit
def _run(x, w):
    mesh = plsc.VectorSubcoreMesh(
        core_axis_name="c", subcore_axis_name="s", num_cores=NC, num_subcores=NS
    )
    return pl.kernel(
        _kernel_body,
        out_type=jax.ShapeDtypeStruct((B, T, D), jnp.float32),
        mesh=mesh,
        scratch_types=(
            [pltpu.VMEM((CPOS, D), jnp.float32)] * (NSLOT + 2)
            + [pltpu.SemaphoreType.DMA] * (2 * NSLOT + 2)
        ),
    )(x, w)


def kernel(x, lookup_weight):
    return _run(x, lookup_weight)


# speedup vs baseline: 1.1412x; 1.0258x over previous
import jax
import jax.numpy as jnp
from jax import lax
from jax.experimental import pallas as pl
from jax.experimental.pallas import tpu as pltpu
from jax.experimental.pallas import tpu_sc as plsc

B, T, D = 4, 8192, 768
NC, NS = 2, 16
NW = NC * NS
PW = T // NW
CPOS = 16
NCHUNK = PW // CPOS
NPAIR = NCHUNK // 2
NSL = D // 16
NSLOT = 8


def _body(x_hbm, w_hbm, out_hbm, refs, sems):
    xb = refs[:NSLOT]
    wb = refs[NSLOT:]
    lsem = sems[:NSLOT]
    ssem = sems[NSLOT:2 * NSLOT]
    wsem = sems[2 * NSLOT:]
    wid = lax.axis_index("s") * NC + lax.axis_index("c")
    base = wid * PW

    def xsl(b, c):
        return x_hbm.at[b, pl.ds(base + c * CPOS, CPOS), :]

    def osl(b, c):
        return out_hbm.at[b, pl.ds(base + c * CPOS, CPOS), :]

    def wslc(c):
        return w_hbm.at[pl.ds(base + c * CPOS, CPOS), :]

    def emit_pair(pair, first, last):
        c0 = 2 * pair
        for u in range(NSLOT):
            b = u % B
            half = u // B
            c = c0 + half
            wbuf = wb[half]
            if u == 0:
                if not first:
                    pltpu.async_copy(wslc(c0 + 1), wb[1], wsem[1])
                pltpu.make_async_copy(wslc(c0), wb[0], wsem[0]).wait()
            if u == B:
                pltpu.make_async_copy(wslc(c0 + 1), wb[1], wsem[1]).wait()
                if not last:
                    pltpu.async_copy(wslc(c0 + 2), wb[0], wsem[0])

            pltpu.make_async_copy(xsl(b, c), xb[u], lsem[u]).wait()

            u4 = (u + 4) % NSLOT
            c4 = c + 1
            if not (last and u >= B):
                if not (first and u < B):
                    cp = c - 1
                    pltpu.make_async_copy(xb[u4], osl(b, cp), ssem[u4]).wait()
                pltpu.async_copy(xsl(b, c4), xb[u4], lsem[u4])

            @plsc.parallel_loop(0, CPOS, 1)
            def rowadd(r):
                @plsc.parallel_loop(0, D, 16, unroll=6)
                def coladd(col):
                    plsc.addupdate(xb[u].at[r, pl.ds(col, 16)],
                                   wb[half][r, pl.ds(col, 16)])

            pltpu.async_copy(xb[u], osl(b, c), ssem[u])

    pltpu.async_copy(wslc(0), wb[0], wsem[0])
    pltpu.async_copy(wslc(1), wb[1], wsem[1])
    for u in range(B):
        pltpu.async_copy(xsl(u, 0), xb[u], lsem[u])

    emit_pair(0, True, NPAIR == 1)
    if NPAIR > 2:
        def loop_body(pair, carry):
            emit_pair(pair, False, False)
            return carry
        lax.fori_loop(1, NPAIR - 1, loop_body, 0)
    if NPAIR > 1:
        emit_pair(NPAIR - 1, False, True)

    for u in range(NSLOT):
        b = u % B
        c = NCHUNK - 2 + u // B
        pltpu.make_async_copy(xb[u], osl(b, c), ssem[u]).wait()


def _kernel_body(x_hbm, w_hbm, out_hbm, *scratch):
    _body(x_hbm, w_hbm, out_hbm, scratch[:NSLOT + 2], scratch[NSLOT + 2:])


@jax.jit
def _run(x, w):
    mesh = plsc.VectorSubcoreMesh(
        core_axis_name="c", subcore_axis_name="s", num_cores=NC, num_subcores=NS
    )
    return pl.kernel(
        _kernel_body,
        out_type=jax.ShapeDtypeStruct((B, T, D), jnp.float32),
        mesh=mesh,
        scratch_types=(
            [pltpu.VMEM((CPOS, D), jnp.float32)] * (NSLOT + 2)
            + [pltpu.SemaphoreType.DMA] * (2 * NSLOT + 2)
        ),
    )(x, w)


def kernel(x, lookup_weight):
    return _run(x, lookup_weight)
